# baseline (speedup 1.0000x reference)
import jax
import jax.numpy as jnp
from jax import lax
from jax.experimental import pallas as pl
from jax.experimental.pallas import tpu as pltpu
from jax.experimental.pallas import tpu_sc as plsc

VOCAB_SIZE = 1_000_000
N_EMBD = 64
SEQ_LEN = 200
BATCH = 1024
NTOK = BATCH * SEQ_LEN

_info = plsc.get_sparse_core_info()
_NC, _NS = _info.num_cores, _info.num_subcores
NW = _NC * _NS
SLAB = 128
NSLAB = 246
STRIPE = NSLAB * SLAB
CAP = 8192
NCH = NTOK // 1024
DUMP = NTOK


def _emb_body(xi_hbm, ttT_hbm, pos_hbm, tailT_hbm, out_hbm,
              xs, r0, t0, r1, t1, slab_v, pos_v, tail_v, obuf, tl_v, bnd,
              xsem, slsem, scsem):
    cid = lax.axis_index("c")
    sid = lax.axis_index("s")
    wid = sid * _NC + cid
    lo = wid * STRIPE
    hi = lo + STRIPE

    pltpu.sync_copy(pos_hbm, pos_v)

    iota = lax.iota(jnp.int32, 16)

    def xstage(ch, b):
        off = pl.multiple_of(ch * 8, 8)
        pltpu.async_copy(xi_hbm.at[pl.ds(off, 8)], xs.at[b], xsem.at[b])

    def xwait(b):
        pltpu.make_async_copy(xi_hbm.at[pl.ds(0, 8)], xs.at[b],
                              xsem.at[b]).wait()

    xstage(0, 0)
    xstage(1, 1)

    def scan_chunk(ch, cur):
        for b in range(2):
            c = ch * 2 + b
            xwait(b)
            for rr in range(8):
                for g in range(8):
                    v = xs[b, rr, pl.ds(g * 16, 16)]
                    m = (v >= lo) & (v < hi)
                    tvec = (c * 1024 + rr * 128 + g * 16) + iota
                    plsc.store_compressed(r0.at[pl.ds(cur, 16)], v - lo, mask=m)
                    plsc.store_compressed(t0.at[pl.ds(cur, 16)], tvec, mask=m)
                    cnt = plsc.all_reduce_population_count(m)
                    cur = cur + cnt[0]

            @pl.when(ch * 2 + b + 2 < NCH)
            def _():
                xstage(c + 2, b)
        return cur

    nmatch = lax.fori_loop(0, NCH // 2, scan_chunk, 0)
    bnd[0, 0] = 0
    bnd[0, 1] = nmatch

    bufs = ((r0, t0), (r1, t1))
    for lev in range(8):
        src, dst = bufs[lev % 2], bufs[(lev + 1) % 2]
        pin, pout = lev % 2, (lev + 1) % 2
        bit = 7 + (7 - lev)

        def seg_body(j, c, src=src, dst=dst, pin=pin, pout=pout, bit=bit):
            start = bnd[pin, j]
            end = bnd[pin, j + 1]

            def grp(g, cs):
                c0, c1 = cs
                p = start + g * 16
                rv = src[0][pl.ds(p, 16)]
                tv = src[1][pl.ds(p, 16)]
                valid = (p + iota) < end
                one = ((rv >> bit) & 1) == 1
                m0 = valid & (~one)
                m1 = valid & one
                n0 = plsc.all_reduce_population_count(m0)[0]
                n1 = plsc.all_reduce_population_count(m1)[0]
                plsc.store_compressed(dst[0].at[pl.ds(c0, 16)], rv, mask=m0)
                plsc.store_compressed(dst[1].at[pl.ds(c0, 16)], tv, mask=m0)
                rvr = lax.rev(rv, (0,))
                tvr = lax.rev(tv, (0,))
                validr = (p + 15 - iota) < end
                m1r = validr & ((((rvr >> bit) & 1)) == 1)
                plsc.store_compressed(dst[0].at[pl.ds(c1 - n1, 16)], rvr, mask=m1r)
                plsc.store_compressed(dst[1].at[pl.ds(c1 - n1, 16)], tvr, mask=m1r)
                return (c0 + n0, c1 - n1)

            ngrp = (end - start + 15) >> 4
            mid, _ = lax.fori_loop(0, ngrp, grp, (start, end))
            bnd[pout, 2 * j] = start
            bnd[pout, 2 * j + 1] = mid
            return c

        lax.fori_loop(0, 1 << lev, seg_body, 0)
        bnd[(lev + 1) % 2, 2 << lev] = nmatch

    def slstage(s, b):
        cb_raw = lo + s * SLAB
        cb = jnp.where(cb_raw + SLAB > VOCAB_SIZE, 0, cb_raw)
        cb = pl.multiple_of(cb, SLAB)
        pltpu.async_copy(ttT_hbm.at[:, pl.ds(cb, SLAB)], slab_v.at[b],
                         slsem.at[b])

    def slwait(b):
        pltpu.make_async_copy(ttT_hbm.at[:, pl.ds(0, SLAB)], slab_v.at[b],
                              slsem.at[b]).wait()

    def scat_wait(b):
        pltpu.make_async_copy(obuf.at[b], out_hbm.at[pl.ds(0, 16)],
                              scsem.at[b]).wait()

    pltpu.sync_copy(tailT_hbm, tail_v)

    dumpvec = jnp.full((16,), DUMP, jnp.int32)
    for bb in range(2):
        tl_v[bb, pl.ds(0, 16)] = dumpvec
        pltpu.async_copy(obuf.at[bb], out_hbm.at[tl_v.at[bb]], scsem.at[bb])

    slstage(0, 0)
    slstage(1, 1)

    def emit_groups(src_ref, coff, start, end):
        def grp(g, c):
            p = start + g * 16
            rv = r0[pl.ds(p, 16)]
            tv = t0[pl.ds(p, 16)]
            valid = (p + iota) < end
            cvec = jnp.clip(rv - coff, 0, src_ref.shape[1] - 1)
            lrow = jnp.clip(lax.rem(tv, SEQ_LEN), 0, SEQ_LEN - 1)
            gpar = lax.rem(g, 2)
            for bb in range(2):
                @pl.when(gpar == bb)
                def _(bb=bb):
                    scat_wait(bb)
                    for d in range(N_EMBD):
                        dfull = jnp.full((16,), d, jnp.int32)
                        vals = plsc.load_gather(src_ref, [dfull, cvec])
                        pvals = plsc.load_gather(pos_v, [lrow, dfull])
                        plsc.store_scatter(
                            obuf.at[bb], [iota, dfull], vals + pvals)
                    tl_v[bb, pl.ds(0, 16)] = jnp.where(valid, tv, DUMP)
                    pltpu.async_copy(obuf.at[bb],
                                     out_hbm.at[tl_v.at[bb]],
                                     scsem.at[bb])
            return c

        ngrp = (end - start + 15) >> 4
        lax.fori_loop(0, ngrp, grp, 0)

    def slab_body(i, carry):
        for b in range(2):
            s = i * 2 + b
            start = bnd[0, s]
            end = bnd[0, s + 1]
            slwait(b)
            cb_raw = lo + s * SLAB
            is_tail = cb_raw + SLAB > VOCAB_SIZE

            @pl.when(jnp.logical_not(is_tail))
            def _(b=b, start=start, end=end, cb_raw=cb_raw):
                emit_groups(slab_v.at[b], cb_raw - lo, start, end)

            @pl.when(is_tail)
            def _(start=start, end=end):
                emit_groups(tail_v, (VOCAB_SIZE - 64) - lo, start, end)

            @pl.when(s + 2 < NSLAB)
            def _():
                slstage(s + 2, b)
        return carry

    lax.fori_loop(0, NSLAB // 2, slab_body, 0)


def kernel(x, token_table, position_table):
    xi = x.astype(jnp.int32).reshape(NTOK // 128, 128)
    ttT = token_table.T
    tailT = token_table[VOCAB_SIZE - 64:].T
    run = pl.kernel(
        _emb_body,
        out_type=jax.ShapeDtypeStruct((NTOK + 16, 2 * N_EMBD), jnp.float32),
        mesh=plsc.VectorSubcoreMesh(core_axis_name="c", subcore_axis_name="s"),
        scratch_types=[
            pltpu.VMEM((2, 8, 128), jnp.int32),
            pltpu.VMEM((CAP,), jnp.int32),
            pltpu.VMEM((CAP,), jnp.int32),
            pltpu.VMEM((CAP,), jnp.int32),
            pltpu.VMEM((CAP,), jnp.int32),
            pltpu.VMEM((2, N_EMBD, SLAB), jnp.float32),
            pltpu.VMEM((SEQ_LEN, N_EMBD), jnp.float32),
            pltpu.VMEM((N_EMBD, 64), jnp.float32),
            pltpu.VMEM((2, 16, 2 * N_EMBD), jnp.float32),
            pltpu.VMEM((2, 16), jnp.int32),
            pltpu.SMEM((2, 520), jnp.int32),
            pltpu.SemaphoreType.DMA((2,)),
            pltpu.SemaphoreType.DMA((2,)),
            pltpu.SemaphoreType.DMA((2,)),
        ],
        compiler_params=pltpu.CompilerParams(use_tc_tiling_on_sc=True,
                                             needs_layout_passes=False),
    )
    out3 = run(xi, ttT, position_table, tailT)
    return out3[:NTOK, :N_EMBD].reshape(BATCH, SEQ_LEN, N_EMBD)
